# baseline (speedup 1.0000x reference)
import math

import jax
import jax.numpy as jnp
from jax import lax
from jax.experimental import pallas as pl
from jax.experimental.pallas import tpu as pltpu
from jax.experimental.pallas import tpu_sc as plsc

NC = 2
NS = 16
L = 16
NW = NC * NS

B = 1024
S = 200
D = 128
N = B * S
RPW = N // NW
BPW = B // NW
CH = 128
NCH = RPW // CH
NJ = D // (2 * L)
SCALE = math.sqrt(D)
BF = jnp.bfloat16


def _body(tok_ids, pos_ids, note_ids, conc_ids, sids, eids, tids,
          tok_tab, note_tab, conc_tab, pos_tab, sea_tab, emo_tab, tim_tab,
          out,
          ids_tok_v, ids_pos_v, ids_note_v, ids_conc_v,
          note_v, conc_v, sea_v, emo_v, tim_v, combo_v, bvec_v,
          sid_v, eid_v, tid_v,
          tok_buf0, tok_buf1, pos_buf0, pos_buf1, out_buf0, out_buf1,
          sem_t0, sem_t1, sem_p0, sem_p1, sem_o0, sem_o1):
    wid = lax.axis_index("s") * NC + lax.axis_index("c")
    row0 = wid * RPW
    c0 = wid * NCH
    bg0 = wid * BPW

    pltpu.sync_copy(tok_ids.at[pl.ds(c0, NCH)], ids_tok_v)
    pltpu.sync_copy(pos_ids.at[pl.ds(c0, NCH)], ids_pos_v)
    pltpu.sync_copy(note_ids.at[pl.ds(c0, NCH)], ids_note_v)
    pltpu.sync_copy(conc_ids.at[pl.ds(c0, NCH)], ids_conc_v)
    pltpu.sync_copy(note_tab, note_v)
    pltpu.sync_copy(conc_tab, conc_v)
    pltpu.sync_copy(sea_tab, sea_v)
    pltpu.sync_copy(emo_tab, emo_v)
    pltpu.sync_copy(tim_tab, tim_v)
    pltpu.sync_copy(sids.at[pl.ds(bg0, BPW)], sid_v)
    pltpu.sync_copy(eids.at[pl.ds(bg0, BPW)], eid_v)
    pltpu.sync_copy(tids.at[pl.ds(bg0, BPW)], tid_v)

    iota = lax.iota(jnp.int32, L)
    wcols = [iota + j * L for j in range(NJ)]
    sceven = [2 * iota + j * 2 * L for j in range(NJ)]
    scodd = [2 * iota + j * 2 * L + 1 for j in range(NJ)]

    def combo_step(c, carry):
        n = c // 20
        k = c - n * 20
        for j in range(NJ):
            s = (note_v[n, pl.ds(j * 2 * L, 2 * L)]
                 + conc_v[k, pl.ds(j * 2 * L, 2 * L)])
            combo_v[c, pl.ds(j * L, L)] = plsc.bitcast(s, jnp.int32)
        return carry
    lax.fori_loop(0, 60, combo_step, 0)

    for g in range(BPW // L):
        sv = sid_v[pl.ds(g * L, L)]
        ev = eid_v[pl.ds(g * L, L)]
        tv = tid_v[pl.ds(g * L, L)]
        for r in range(L):
            for j in range(NJ):
                bvec_v[g * L + r, pl.ds(j * 2 * L, 2 * L)] = (
                    sea_v[sv[r], pl.ds(j * 2 * L, 2 * L)]
                    + emo_v[ev[r], pl.ds(j * 2 * L, 2 * L)]
                    + tim_v[tv[r], pl.ds(j * 2 * L, 2 * L)])

    def compute_chunk(c, tok_b, pos_b, out_b):
        r0 = c * CH
        bA = r0 // S
        mid = jnp.minimum((bA + 1) * S - r0, CH)
        bB = jnp.minimum(bA + 1, BPW - 1)
        bvA = [bvec_v[bA, pl.ds(j * 2 * L, 2 * L)] for j in range(NJ)]
        bvB = [bvec_v[bB, pl.ds(j * 2 * L, 2 * L)] for j in range(NJ)]

        @plsc.parallel_loop(0, CH // L, step=1)
        def h_step(h):
            nidv = ids_note_v[c, pl.ds(h * L, L)]
            cidv = ids_conc_v[c, pl.ds(h * L, L)]
            cxv = nidv * 20 + cidv
            base = h * L
            for r in range(L):
                i = base + r
                ridx = jnp.full((L,), cxv[r], jnp.int32)
                rsplat = jnp.full((L,), i, jnp.int32)
                pred = i >= mid
                for j in range(NJ):
                    tv = plsc.bitcast(tok_b[i, pl.ds(j * L, L)], BF)
                    pv = plsc.bitcast(pos_b[i, pl.ds(j * L, L)], BF)
                    cw = plsc.load_gather(combo_v, [ridx, wcols[j]])
                    cv = plsc.bitcast(cw, BF)
                    bvj = jnp.where(pred, bvB[j], bvA[j])
                    acc = (tv * SCALE + pv) + (cv + bvj)
                    w = plsc.bitcast(acc, jnp.int32)
                    fe = plsc.bitcast(w << 16, jnp.float32)
                    fo = plsc.bitcast(w & (-65536), jnp.float32)
                    plsc.store_scatter(out_b, [rsplat, sceven[j]], fe)
                    plsc.store_scatter(out_b, [rsplat, scodd[j]], fo)

    def g_tok(c, buf, sem):
        pltpu.async_copy(tok_tab.at[ids_tok_v.at[c]], buf, sem)

    def g_pos(c, buf, sem):
        pltpu.async_copy(pos_tab.at[ids_pos_v.at[c]], buf, sem)

    def w_tok(c, buf, sem):
        pltpu.make_async_copy(tok_tab.at[ids_tok_v.at[c]], buf, sem).wait()

    def w_pos(c, buf, sem):
        pltpu.make_async_copy(pos_tab.at[ids_pos_v.at[c]], buf, sem).wait()

    def put_out(c, buf, sem):
        pltpu.async_copy(buf, out.at[pl.ds(row0 + c * CH, CH)], sem)

    def w_out(buf, sem):
        pltpu.make_async_copy(out.at[pl.ds(row0, CH)], buf, sem).wait()

    g_tok(0, tok_buf0, sem_t0)
    g_pos(0, pos_buf0, sem_p0)
    g_tok(1, tok_buf1, sem_t1)
    g_pos(1, pos_buf1, sem_p1)
    w_tok(0, tok_buf0, sem_t0)
    w_pos(0, pos_buf0, sem_p0)
    compute_chunk(0, tok_buf0, pos_buf0, out_buf0)
    put_out(0, out_buf0, sem_o0)

    def pipe_step(cc, carry):
        c1 = 2 * cc + 1
        c2 = 2 * cc + 2
        w_tok(c1, tok_buf1, sem_t1)
        w_pos(c1, pos_buf1, sem_p1)
        w_out(out_buf0, sem_o0)
        g_tok(c2, tok_buf0, sem_t0)
        g_pos(c2, pos_buf0, sem_p0)
        compute_chunk(c1, tok_buf1, pos_buf1, out_buf1)
        put_out(c1, out_buf1, sem_o1)
        c3 = c2 + 1
        w_tok(c2, tok_buf0, sem_t0)
        w_pos(c2, pos_buf0, sem_p0)
        w_out(out_buf1, sem_o1)
        g_tok(c3, tok_buf1, sem_t1)
        g_pos(c3, pos_buf1, sem_p1)
        compute_chunk(c2, tok_buf0, pos_buf0, out_buf0)
        put_out(c2, out_buf0, sem_o0)
        return carry
    lax.fori_loop(0, (NCH - 2) // 2, pipe_step, 0)

    cl = NCH - 1
    w_tok(cl, tok_buf1, sem_t1)
    w_pos(cl, pos_buf1, sem_p1)
    compute_chunk(cl, tok_buf1, pos_buf1, out_buf1)
    put_out(cl, out_buf1, sem_o1)
    w_out(out_buf0, sem_o0)
    w_out(out_buf1, sem_o1)


def kernel(input_ids, note_type_ids, concentration_ids, position_ids,
           season_ids, emotion_ids, time_ids,
           token_table, note_table, conc_table, pos_table,
           season_table, emotion_table, time_table):
    tok_ids = input_ids.reshape(N // CH, CH).astype(jnp.int32)
    pos_ids2 = position_ids.reshape(N // CH, CH).astype(jnp.int32)
    note_ids2 = note_type_ids.reshape(N // CH, CH).astype(jnp.int32)
    conc_ids2 = concentration_ids.reshape(N // CH, CH).astype(jnp.int32)
    sids = season_ids.astype(jnp.int32)
    eids = emotion_ids.astype(jnp.int32)
    tids = time_ids.astype(jnp.int32)

    mesh = plsc.VectorSubcoreMesh(
        core_axis_name="c", subcore_axis_name="s",
        num_cores=NC, num_subcores=NS)
    run = pl.kernel(
        _body,
        out_type=jax.ShapeDtypeStruct((N, D), jnp.float32),
        mesh=mesh,
        compiler_params=pltpu.CompilerParams(
            use_tc_tiling_on_sc=False, needs_layout_passes=False),
        scratch_types=[
            pltpu.VMEM((NCH, CH), jnp.int32),
            pltpu.VMEM((NCH, CH), jnp.int32),
            pltpu.VMEM((NCH, CH), jnp.int32),
            pltpu.VMEM((NCH, CH), jnp.int32),
            pltpu.VMEM((3, D), BF),
            pltpu.VMEM((20, D), BF),
            pltpu.VMEM((4, D), BF),
            pltpu.VMEM((8, D), BF),
            pltpu.VMEM((4, D), BF),
            pltpu.VMEM((60, D // 2), jnp.int32),
            pltpu.VMEM((BPW, D), BF),
            pltpu.VMEM((BPW,), jnp.int32),
            pltpu.VMEM((BPW,), jnp.int32),
            pltpu.VMEM((BPW,), jnp.int32),
            pltpu.VMEM((CH, D // 2), jnp.int32),
            pltpu.VMEM((CH, D // 2), jnp.int32),
            pltpu.VMEM((CH, D // 2), jnp.int32),
            pltpu.VMEM((CH, D // 2), jnp.int32),
            pltpu.VMEM((CH, D), jnp.float32),
            pltpu.VMEM((CH, D), jnp.float32),
            pltpu.SemaphoreType.DMA,
            pltpu.SemaphoreType.DMA,
            pltpu.SemaphoreType.DMA,
            pltpu.SemaphoreType.DMA,
            pltpu.SemaphoreType.DMA,
            pltpu.SemaphoreType.DMA,
        ],
    )
    def pack_i32(t):
        tb = t.astype(BF)
        return lax.bitcast_convert_type(
            tb.reshape(t.shape[0], t.shape[1] // 2, 2), jnp.int32)

    out = run(tok_ids, pos_ids2, note_ids2, conc_ids2, sids, eids, tids,
              pack_i32(token_table), note_table.astype(BF),
              conc_table.astype(BF), pack_i32(pos_table),
              season_table.astype(BF), emotion_table.astype(BF),
              time_table.astype(BF))
    return out.reshape(B, S, D)


# speedup vs baseline: 2.3460x; 2.3460x over previous
import math

import jax
import jax.numpy as jnp
from jax import lax
from jax.experimental import pallas as pl
from jax.experimental.pallas import tpu as pltpu
from jax.experimental.pallas import tpu_sc as plsc

NC = 2
NS = 16
L = 16
NW = NC * NS

B = 1024
S = 200
D = 128
N = B * S
RPW = N // NW
BPW = B // NW
CH = 128
NCH = RPW // CH
NJ = D // (2 * L)
SCALE = math.sqrt(D)
BF = jnp.bfloat16


def _body(tok_ids, pos_ids, note_ids, conc_ids, sids, eids, tids,
          tok_tab, note_tab, conc_tab, pos_tab, sea_tab, emo_tab, tim_tab,
          out,
          ids_tok_v, ids_pos_v, ids_note_v, ids_conc_v,
          note_v, conc_v, sea_v, emo_v, tim_v, combo_v, bvec_v,
          sid_v, eid_v, tid_v,
          tok_buf0, tok_buf1, pos_buf0, pos_buf1,
          sem_t0, sem_t1, sem_p0, sem_p1, sem_o0, sem_o1):
    wid = lax.axis_index("s") * NC + lax.axis_index("c")
    row0 = wid * RPW
    c0 = wid * NCH
    bg0 = wid * BPW

    pltpu.sync_copy(tok_ids.at[pl.ds(c0, NCH)], ids_tok_v)
    pltpu.sync_copy(pos_ids.at[pl.ds(c0, NCH)], ids_pos_v)
    pltpu.sync_copy(note_ids.at[pl.ds(c0, NCH)], ids_note_v)
    pltpu.sync_copy(conc_ids.at[pl.ds(c0, NCH)], ids_conc_v)
    pltpu.sync_copy(note_tab, note_v)
    pltpu.sync_copy(conc_tab, conc_v)
    pltpu.sync_copy(sea_tab, sea_v)
    pltpu.sync_copy(emo_tab, emo_v)
    pltpu.sync_copy(tim_tab, tim_v)
    pltpu.sync_copy(sids.at[pl.ds(bg0, BPW)], sid_v)
    pltpu.sync_copy(eids.at[pl.ds(bg0, BPW)], eid_v)
    pltpu.sync_copy(tids.at[pl.ds(bg0, BPW)], tid_v)

    iota = lax.iota(jnp.int32, L)
    wcols = [iota + j * L for j in range(NJ)]

    def combo_step(c, carry):
        n = c // 20
        k = c - n * 20
        for j in range(NJ):
            s = (note_v[n, pl.ds(j * 2 * L, 2 * L)]
                 + conc_v[k, pl.ds(j * 2 * L, 2 * L)])
            combo_v[c, pl.ds(j * L, L)] = plsc.bitcast(s, jnp.int32)
        return carry
    lax.fori_loop(0, 60, combo_step, 0)

    for g in range(BPW // L):
        sv = sid_v[pl.ds(g * L, L)]
        ev = eid_v[pl.ds(g * L, L)]
        tv = tid_v[pl.ds(g * L, L)]
        for r in range(L):
            for j in range(NJ):
                bvec_v[g * L + r, pl.ds(j * 2 * L, 2 * L)] = (
                    sea_v[sv[r], pl.ds(j * 2 * L, 2 * L)]
                    + emo_v[ev[r], pl.ds(j * 2 * L, 2 * L)]
                    + tim_v[tv[r], pl.ds(j * 2 * L, 2 * L)])

    def compute_chunk(c, tok_b, pos_b):
        r0 = c * CH
        bA = r0 // S
        mid = jnp.minimum((bA + 1) * S - r0, CH)
        bB = jnp.minimum(bA + 1, BPW - 1)
        bvA = [bvec_v[bA, pl.ds(j * 2 * L, 2 * L)] for j in range(NJ)]
        bvB = [bvec_v[bB, pl.ds(j * 2 * L, 2 * L)] for j in range(NJ)]

        @plsc.parallel_loop(0, CH // L, step=1)
        def h_step(h):
            nidv = ids_note_v[c, pl.ds(h * L, L)]
            cidv = ids_conc_v[c, pl.ds(h * L, L)]
            cxv = nidv * 20 + cidv
            base = h * L
            for r in range(L):
                i = base + r
                ridx = jnp.full((L,), cxv[r], jnp.int32)
                pred = i >= mid
                for j in range(NJ):
                    pv = plsc.bitcast(pos_b[i, pl.ds(j * L, L)], BF)
                    cw = plsc.load_gather(combo_v, [ridx, wcols[j]])
                    cv = plsc.bitcast(cw, BF)
                    bvj = jnp.where(pred, bvB[j], bvA[j])
                    side = (pv + cv) + bvj
                    w = plsc.bitcast(side, jnp.int32)
                    flo = plsc.bitcast(w << 16, jnp.float32)
                    fhi = plsc.bitcast(w & (-65536), jnp.float32)
                    tlo = tok_b[i, pl.ds(j * 2 * L, L)]
                    thi = tok_b[i, pl.ds(j * 2 * L + L, L)]
                    tok_b[i, pl.ds(j * 2 * L, L)] = tlo * SCALE + flo
                    tok_b[i, pl.ds(j * 2 * L + L, L)] = thi * SCALE + fhi

    def g_tok(c, buf, sem):
        pltpu.async_copy(tok_tab.at[ids_tok_v.at[c]], buf, sem)

    def g_pos(c, buf, sem):
        pltpu.async_copy(pos_tab.at[ids_pos_v.at[c]], buf, sem)

    def w_tok(c, buf, sem):
        pltpu.make_async_copy(tok_tab.at[ids_tok_v.at[c]], buf, sem).wait()

    def w_pos(c, buf, sem):
        pltpu.make_async_copy(pos_tab.at[ids_pos_v.at[c]], buf, sem).wait()

    def put_out(c, buf, sem):
        pltpu.async_copy(buf, out.at[pl.ds(row0 + c * CH, CH)], sem)

    def w_out(buf, sem):
        pltpu.make_async_copy(out.at[pl.ds(row0, CH)], buf, sem).wait()

    g_tok(0, tok_buf0, sem_t0)
    g_pos(0, pos_buf0, sem_p0)
    g_tok(1, tok_buf1, sem_t1)
    g_pos(1, pos_buf1, sem_p1)
    w_tok(0, tok_buf0, sem_t0)
    w_pos(0, pos_buf0, sem_p0)
    compute_chunk(0, tok_buf0, pos_buf0)
    put_out(0, tok_buf0, sem_o0)

    def pipe_step(cc, carry):
        c1 = 2 * cc + 1
        c2 = 2 * cc + 2
        w_tok(c1, tok_buf1, sem_t1)
        w_pos(c1, pos_buf1, sem_p1)
        w_out(tok_buf0, sem_o0)
        g_tok(c2, tok_buf0, sem_t0)
        g_pos(c2, pos_buf0, sem_p0)
        compute_chunk(c1, tok_buf1, pos_buf1)
        put_out(c1, tok_buf1, sem_o1)
        c3 = c2 + 1
        w_tok(c2, tok_buf0, sem_t0)
        w_pos(c2, pos_buf0, sem_p0)
        w_out(tok_buf1, sem_o1)
        g_tok(c3, tok_buf1, sem_t1)
        g_pos(c3, pos_buf1, sem_p1)
        compute_chunk(c2, tok_buf0, pos_buf0)
        put_out(c2, tok_buf0, sem_o0)
        return carry
    lax.fori_loop(0, (NCH - 2) // 2, pipe_step, 0)

    cl = NCH - 1
    w_tok(cl, tok_buf1, sem_t1)
    w_pos(cl, pos_buf1, sem_p1)
    compute_chunk(cl, tok_buf1, pos_buf1)
    put_out(cl, tok_buf1, sem_o1)
    w_out(tok_buf0, sem_o0)
    w_out(tok_buf1, sem_o1)


def kernel(input_ids, note_type_ids, concentration_ids, position_ids,
           season_ids, emotion_ids, time_ids,
           token_table, note_table, conc_table, pos_table,
           season_table, emotion_table, time_table):
    tok_ids = input_ids.reshape(N // CH, CH).astype(jnp.int32)
    pos_ids2 = position_ids.reshape(N // CH, CH).astype(jnp.int32)
    note_ids2 = note_type_ids.reshape(N // CH, CH).astype(jnp.int32)
    conc_ids2 = concentration_ids.reshape(N // CH, CH).astype(jnp.int32)
    sids = season_ids.astype(jnp.int32)
    eids = emotion_ids.astype(jnp.int32)
    tids = time_ids.astype(jnp.int32)

    mesh = plsc.VectorSubcoreMesh(
        core_axis_name="c", subcore_axis_name="s",
        num_cores=NC, num_subcores=NS)
    run = pl.kernel(
        _body,
        out_type=jax.ShapeDtypeStruct((N, D), jnp.float32),
        mesh=mesh,
        compiler_params=pltpu.CompilerParams(
            use_tc_tiling_on_sc=False, needs_layout_passes=False),
        scratch_types=[
            pltpu.VMEM((NCH, CH), jnp.int32),
            pltpu.VMEM((NCH, CH), jnp.int32),
            pltpu.VMEM((NCH, CH), jnp.int32),
            pltpu.VMEM((NCH, CH), jnp.int32),
            pltpu.VMEM((3, D), BF),
            pltpu.VMEM((20, D), BF),
            pltpu.VMEM((4, D), BF),
            pltpu.VMEM((8, D), BF),
            pltpu.VMEM((4, D), BF),
            pltpu.VMEM((60, D // 2), jnp.int32),
            pltpu.VMEM((BPW, D), BF),
            pltpu.VMEM((BPW,), jnp.int32),
            pltpu.VMEM((BPW,), jnp.int32),
            pltpu.VMEM((BPW,), jnp.int32),
            pltpu.VMEM((CH, D), jnp.float32),
            pltpu.VMEM((CH, D), jnp.float32),
            pltpu.VMEM((CH, D // 2), jnp.int32),
            pltpu.VMEM((CH, D // 2), jnp.int32),
            pltpu.SemaphoreType.DMA,
            pltpu.SemaphoreType.DMA,
            pltpu.SemaphoreType.DMA,
            pltpu.SemaphoreType.DMA,
            pltpu.SemaphoreType.DMA,
            pltpu.SemaphoreType.DMA,
        ],
    )
    perm = (jnp.arange(0, D, 2 * L)[:, None]
            + jnp.stack([jnp.arange(L), jnp.arange(L) + L], 1).reshape(-1)
            ).reshape(-1)

    def pack_i32(t):
        tb = t[:, perm].astype(BF)
        return lax.bitcast_convert_type(
            tb.reshape(t.shape[0], t.shape[1] // 2, 2), jnp.int32)

    def perm_bf(t):
        return t[:, perm].astype(BF)

    out = run(tok_ids, pos_ids2, note_ids2, conc_ids2, sids, eids, tids,
              token_table, perm_bf(note_table),
              perm_bf(conc_table), pack_i32(pos_table),
              perm_bf(season_table), perm_bf(emotion_table),
              perm_bf(time_table))
    return out.reshape(B, S, D)


# speedup vs baseline: 3.9612x; 1.6885x over previous
import math

import jax
import jax.numpy as jnp
from jax import lax
from jax.experimental import pallas as pl
from jax.experimental.pallas import tpu as pltpu
from jax.experimental.pallas import tpu_sc as plsc

NC = 2
NS = 16
L = 16
NW = NC * NS

B = 1024
S = 200
D = 128
N = B * S
RPW = N // NW
BPW = B // NW
CH = 128
NCH = RPW // CH
NJ = D // (2 * L)
SCALE = math.sqrt(D)
BF = jnp.bfloat16


def _body(tok_ids, pos_ids, note_ids, conc_ids, sids, eids, tids,
          tok_tab, note_tab, conc_tab, pos_tab, sea_tab, emo_tab, tim_tab,
          out,
          ids_tok_v, ids_pos_v, ids_note_v, ids_conc_v,
          note_v, conc_v, sea_v, emo_v, tim_v, combo_v, bvec_v,
          sid_v, eid_v, tid_v,
          tok_buf0, tok_buf1, pos_buf0, pos_buf1,
          sem_t0, sem_t1, sem_p0, sem_p1, sem_o0, sem_o1):
    wid = lax.axis_index("s") * NC + lax.axis_index("c")
    row0 = wid * RPW
    c0 = wid * NCH
    bg0 = wid * BPW

    pltpu.sync_copy(tok_ids.at[pl.ds(c0, NCH)], ids_tok_v)
    pltpu.sync_copy(pos_ids.at[pl.ds(c0, NCH)], ids_pos_v)
    pltpu.sync_copy(note_ids.at[pl.ds(c0, NCH)], ids_note_v)
    pltpu.sync_copy(conc_ids.at[pl.ds(c0, NCH)], ids_conc_v)
    pltpu.sync_copy(note_tab, note_v)
    pltpu.sync_copy(conc_tab, conc_v)
    pltpu.sync_copy(sea_tab, sea_v)
    pltpu.sync_copy(emo_tab, emo_v)
    pltpu.sync_copy(tim_tab, tim_v)
    pltpu.sync_copy(sids.at[pl.ds(bg0, BPW)], sid_v)
    pltpu.sync_copy(eids.at[pl.ds(bg0, BPW)], eid_v)
    pltpu.sync_copy(tids.at[pl.ds(bg0, BPW)], tid_v)

    iota = lax.iota(jnp.int32, L)
    wcols = [iota + j * L for j in range(NJ)]

    def combo_step(c, carry):
        n = c // 20
        k = c - n * 20
        for j in range(NJ):
            s = (note_v[n, pl.ds(j * 2 * L, 2 * L)]
                 + conc_v[k, pl.ds(j * 2 * L, 2 * L)])
            combo_v[c, pl.ds(j * L, L)] = plsc.bitcast(s, jnp.int32)
        return carry
    lax.fori_loop(0, 60, combo_step, 0)

    for g in range(BPW // L):
        sv = sid_v[pl.ds(g * L, L)]
        ev = eid_v[pl.ds(g * L, L)]
        tv = tid_v[pl.ds(g * L, L)]
        for r in range(L):
            for j in range(NJ):
                bvec_v[g * L + r, pl.ds(j * 2 * L, 2 * L)] = (
                    sea_v[sv[r], pl.ds(j * 2 * L, 2 * L)]
                    + emo_v[ev[r], pl.ds(j * 2 * L, 2 * L)]
                    + tim_v[tv[r], pl.ds(j * 2 * L, 2 * L)])

    def compute_chunk(c, tok_b, pos_b):
        r0 = c * CH
        bA = r0 // S
        mid = jnp.minimum((bA + 1) * S - r0, CH)
        bB = jnp.minimum(bA + 1, BPW - 1)
        bvA = [bvec_v[bA, pl.ds(j * 2 * L, 2 * L)] for j in range(NJ)]
        bvB = [bvec_v[bB, pl.ds(j * 2 * L, 2 * L)] for j in range(NJ)]

        @plsc.parallel_loop(0, CH // L, step=1)
        def h_step(h):
            nidv = ids_note_v[c, pl.ds(h * L, L)]
            cidv = ids_conc_v[c, pl.ds(h * L, L)]
            cxv = nidv * 20 + cidv
            base = h * L
            NR = 8
            for r in range(0, L, NR):
                iR = [base + r + q for q in range(NR)]
                ridx = [jnp.full((L,), cxv[r + q], jnp.int32)
                        for q in range(NR)]
                pred = [iR[q] >= mid for q in range(NR)]
                for j in range(NJ):
                    pv = [plsc.bitcast(pos_b[iR[q], pl.ds(j * L, L)], BF)
                          for q in range(NR)]
                    cw = [plsc.load_gather(combo_v, [ridx[q], wcols[j]])
                          for q in range(NR)]
                    tlo = [tok_b[iR[q], pl.ds(j * 2 * L, L)]
                           for q in range(NR)]
                    thi = [tok_b[iR[q], pl.ds(j * 2 * L + L, L)]
                           for q in range(NR)]
                    bvj = [jnp.where(pred[q], bvB[j], bvA[j])
                           for q in range(NR)]
                    w = [plsc.bitcast((plsc.bitcast(cw[q], BF) + pv[q])
                                      + bvj[q], jnp.int32)
                         for q in range(NR)]
                    flo = [plsc.bitcast(w[q] << 16, jnp.float32)
                           for q in range(NR)]
                    fhi = [plsc.bitcast(w[q] & (-65536), jnp.float32)
                           for q in range(NR)]
                    for q in range(NR):
                        tok_b[iR[q], pl.ds(j * 2 * L, L)] = (
                            tlo[q] * SCALE + flo[q])
                    for q in range(NR):
                        tok_b[iR[q], pl.ds(j * 2 * L + L, L)] = (
                            thi[q] * SCALE + fhi[q])

    def g_tok(c, buf, sem):
        pltpu.async_copy(tok_tab.at[ids_tok_v.at[c]], buf, sem)

    def g_pos(c, buf, sem):
        pltpu.async_copy(pos_tab.at[ids_pos_v.at[c]], buf, sem)

    def w_tok(c, buf, sem):
        pltpu.make_async_copy(tok_tab.at[ids_tok_v.at[c]], buf, sem).wait()

    def w_pos(c, buf, sem):
        pltpu.make_async_copy(pos_tab.at[ids_pos_v.at[c]], buf, sem).wait()

    def put_out(c, buf, sem):
        pltpu.async_copy(buf, out.at[pl.ds(row0 + c * CH, CH)], sem)

    def w_out(buf, sem):
        pltpu.make_async_copy(out.at[pl.ds(row0, CH)], buf, sem).wait()

    g_tok(0, tok_buf0, sem_t0)
    g_pos(0, pos_buf0, sem_p0)
    g_tok(1, tok_buf1, sem_t1)
    g_pos(1, pos_buf1, sem_p1)
    w_tok(0, tok_buf0, sem_t0)
    w_pos(0, pos_buf0, sem_p0)
    compute_chunk(0, tok_buf0, pos_buf0)
    put_out(0, tok_buf0, sem_o0)

    def pipe_step(cc, carry):
        c1 = 2 * cc + 1
        c2 = 2 * cc + 2
        w_tok(c1, tok_buf1, sem_t1)
        w_pos(c1, pos_buf1, sem_p1)
        w_out(tok_buf0, sem_o0)
        g_tok(c2, tok_buf0, sem_t0)
        g_pos(c2, pos_buf0, sem_p0)
        compute_chunk(c1, tok_buf1, pos_buf1)
        put_out(c1, tok_buf1, sem_o1)
        c3 = c2 + 1
        w_tok(c2, tok_buf0, sem_t0)
        w_pos(c2, pos_buf0, sem_p0)
        w_out(tok_buf1, sem_o1)
        g_tok(c3, tok_buf1, sem_t1)
        g_pos(c3, pos_buf1, sem_p1)
        compute_chunk(c2, tok_buf0, pos_buf0)
        put_out(c2, tok_buf0, sem_o0)
        return carry
    lax.fori_loop(0, (NCH - 2) // 2, pipe_step, 0)

    cl = NCH - 1
    w_tok(cl, tok_buf1, sem_t1)
    w_pos(cl, pos_buf1, sem_p1)
    compute_chunk(cl, tok_buf1, pos_buf1)
    put_out(cl, tok_buf1, sem_o1)
    w_out(tok_buf0, sem_o0)
    w_out(tok_buf1, sem_o1)


def kernel(input_ids, note_type_ids, concentration_ids, position_ids,
           season_ids, emotion_ids, time_ids,
           token_table, note_table, conc_table, pos_table,
           season_table, emotion_table, time_table):
    tok_ids = input_ids.reshape(N // CH, CH).astype(jnp.int32)
    pos_ids2 = position_ids.reshape(N // CH, CH).astype(jnp.int32)
    note_ids2 = note_type_ids.reshape(N // CH, CH).astype(jnp.int32)
    conc_ids2 = concentration_ids.reshape(N // CH, CH).astype(jnp.int32)
    sids = season_ids.astype(jnp.int32)
    eids = emotion_ids.astype(jnp.int32)
    tids = time_ids.astype(jnp.int32)

    mesh = plsc.VectorSubcoreMesh(
        core_axis_name="c", subcore_axis_name="s",
        num_cores=NC, num_subcores=NS)
    run = pl.kernel(
        _body,
        out_type=jax.ShapeDtypeStruct((N, D), jnp.float32),
        mesh=mesh,
        compiler_params=pltpu.CompilerParams(
            use_tc_tiling_on_sc=False, needs_layout_passes=False),
        scratch_types=[
            pltpu.VMEM((NCH, CH), jnp.int32),
            pltpu.VMEM((NCH, CH), jnp.int32),
            pltpu.VMEM((NCH, CH), jnp.int32),
            pltpu.VMEM((NCH, CH), jnp.int32),
            pltpu.VMEM((3, D), BF),
            pltpu.VMEM((20, D), BF),
            pltpu.VMEM((4, D), BF),
            pltpu.VMEM((8, D), BF),
            pltpu.VMEM((4, D), BF),
            pltpu.VMEM((60, D // 2), jnp.int32),
            pltpu.VMEM((BPW, D), BF),
            pltpu.VMEM((BPW,), jnp.int32),
            pltpu.VMEM((BPW,), jnp.int32),
            pltpu.VMEM((BPW,), jnp.int32),
            pltpu.VMEM((CH, D), jnp.float32),
            pltpu.VMEM((CH, D), jnp.float32),
            pltpu.VMEM((CH, D // 2), jnp.int32),
            pltpu.VMEM((CH, D // 2), jnp.int32),
            pltpu.SemaphoreType.DMA,
            pltpu.SemaphoreType.DMA,
            pltpu.SemaphoreType.DMA,
            pltpu.SemaphoreType.DMA,
            pltpu.SemaphoreType.DMA,
            pltpu.SemaphoreType.DMA,
        ],
    )
    perm = (jnp.arange(0, D, 2 * L)[:, None]
            + jnp.stack([jnp.arange(L), jnp.arange(L) + L], 1).reshape(-1)
            ).reshape(-1)

    def pack_i32(t):
        tb = t[:, perm].astype(BF)
        return lax.bitcast_convert_type(
            tb.reshape(t.shape[0], t.shape[1] // 2, 2), jnp.int32)

    def perm_bf(t):
        return t[:, perm].astype(BF)

    out = run(tok_ids, pos_ids2, note_ids2, conc_ids2, sids, eids, tids,
              token_table, perm_bf(note_table),
              perm_bf(conc_table), pack_i32(pos_table),
              perm_bf(season_table), perm_bf(emotion_table),
              perm_bf(time_table))
    return out.reshape(B, S, D)


# speedup vs baseline: 4.1271x; 1.0419x over previous
import math

import jax
import jax.numpy as jnp
from jax import lax
from jax.experimental import pallas as pl
from jax.experimental.pallas import tpu as pltpu
from jax.experimental.pallas import tpu_sc as plsc

NC = 2
NS = 16
L = 16
NW = NC * NS

B = 1024
S = 200
D = 128
N = B * S
RPW = N // NW
BPW = B // NW
CH = 128
NCH = RPW // CH
NJ = D // (2 * L)
SCALE = math.sqrt(D)
BF = jnp.bfloat16


def _body(tok_ids, pos_ids, note_ids, conc_ids, sids, eids, tids,
          tok_tab, note_tab, conc_tab, pos_tab, sea_tab, emo_tab, tim_tab,
          out,
          ids_tok_v, ids_pos_v, ids_note_v, ids_conc_v,
          note_v, conc_v, sea_v, emo_v, tim_v, combo_v, bvec_v,
          sid_v, eid_v, tid_v,
          pos_res_v, tok_buf0, tok_buf1,
          sem_t0, sem_t1, sem_o0, sem_o1):
    wid = lax.axis_index("s") * NC + lax.axis_index("c")
    row0 = wid * RPW
    c0 = wid * NCH
    bg0 = wid * BPW

    pltpu.sync_copy(tok_ids.at[pl.ds(c0, NCH)], ids_tok_v)
    pltpu.sync_copy(pos_ids.at[pl.ds(c0, NCH)], ids_pos_v)
    pltpu.sync_copy(note_ids.at[pl.ds(c0, NCH)], ids_note_v)
    pltpu.sync_copy(conc_ids.at[pl.ds(c0, NCH)], ids_conc_v)
    pltpu.sync_copy(pos_tab, pos_res_v)
    pltpu.sync_copy(note_tab, note_v)
    pltpu.sync_copy(conc_tab, conc_v)
    pltpu.sync_copy(sea_tab, sea_v)
    pltpu.sync_copy(emo_tab, emo_v)
    pltpu.sync_copy(tim_tab, tim_v)
    pltpu.sync_copy(sids.at[pl.ds(bg0, BPW)], sid_v)
    pltpu.sync_copy(eids.at[pl.ds(bg0, BPW)], eid_v)
    pltpu.sync_copy(tids.at[pl.ds(bg0, BPW)], tid_v)

    iota = lax.iota(jnp.int32, L)
    wcols = [iota + j * L for j in range(NJ)]

    def combo_step(c, carry):
        n = c // 20
        k = c - n * 20
        for j in range(NJ):
            s = (note_v[n, pl.ds(j * 2 * L, 2 * L)]
                 + conc_v[k, pl.ds(j * 2 * L, 2 * L)])
            combo_v[c, pl.ds(j * L, L)] = plsc.bitcast(s, jnp.int32)
        return carry
    lax.fori_loop(0, 60, combo_step, 0)

    for g in range(BPW // L):
        sv = sid_v[pl.ds(g * L, L)]
        ev = eid_v[pl.ds(g * L, L)]
        tv = tid_v[pl.ds(g * L, L)]
        for r in range(L):
            for j in range(NJ):
                bvec_v[g * L + r, pl.ds(j * 2 * L, 2 * L)] = (
                    sea_v[sv[r], pl.ds(j * 2 * L, 2 * L)]
                    + emo_v[ev[r], pl.ds(j * 2 * L, 2 * L)]
                    + tim_v[tv[r], pl.ds(j * 2 * L, 2 * L)])

    def compute_chunk(c, tok_b):
        r0 = c * CH
        bA = r0 // S
        mid = jnp.minimum((bA + 1) * S - r0, CH)
        bB = jnp.minimum(bA + 1, BPW - 1)
        bvA = [bvec_v[bA, pl.ds(j * 2 * L, 2 * L)] for j in range(NJ)]
        bvB = [bvec_v[bB, pl.ds(j * 2 * L, 2 * L)] for j in range(NJ)]

        @plsc.parallel_loop(0, CH // L, step=1)
        def h_step(h):
            nidv = ids_note_v[c, pl.ds(h * L, L)]
            cidv = ids_conc_v[c, pl.ds(h * L, L)]
            pidv = ids_pos_v[c, pl.ds(h * L, L)]
            cxv = nidv * 20 + cidv
            base = h * L
            NR = 8
            for r in range(0, L, NR):
                iR = [base + r + q for q in range(NR)]
                ridx = [jnp.full((L,), cxv[r + q], jnp.int32)
                        for q in range(NR)]
                pidx = [jnp.full((L,), pidv[r + q], jnp.int32)
                        for q in range(NR)]
                pred = [iR[q] >= mid for q in range(NR)]
                for j in range(NJ):
                    pv = [plsc.bitcast(
                              plsc.load_gather(pos_res_v,
                                               [pidx[q], wcols[j]]), BF)
                          for q in range(NR)]
                    cw = [plsc.load_gather(combo_v, [ridx[q], wcols[j]])
                          for q in range(NR)]
                    tlo = [tok_b[iR[q], pl.ds(j * 2 * L, L)]
                           for q in range(NR)]
                    thi = [tok_b[iR[q], pl.ds(j * 2 * L + L, L)]
                           for q in range(NR)]
                    bvj = [jnp.where(pred[q], bvB[j], bvA[j])
                           for q in range(NR)]
                    w = [plsc.bitcast((plsc.bitcast(cw[q], BF) + pv[q])
                                      + bvj[q], jnp.int32)
                         for q in range(NR)]
                    flo = [plsc.bitcast(w[q] << 16, jnp.float32)
                           for q in range(NR)]
                    fhi = [plsc.bitcast(w[q] & (-65536), jnp.float32)
                           for q in range(NR)]
                    for q in range(NR):
                        tok_b[iR[q], pl.ds(j * 2 * L, L)] = (
                            tlo[q] * SCALE + flo[q])
                    for q in range(NR):
                        tok_b[iR[q], pl.ds(j * 2 * L + L, L)] = (
                            thi[q] * SCALE + fhi[q])

    def g_tok(c, buf, sem):
        pltpu.async_copy(tok_tab.at[ids_tok_v.at[c]], buf, sem)

    def w_tok(c, buf, sem):
        pltpu.make_async_copy(tok_tab.at[ids_tok_v.at[c]], buf, sem).wait()

    def put_out(c, buf, sem):
        pltpu.async_copy(buf, out.at[pl.ds(row0 + c * CH, CH)], sem)

    def w_out(buf, sem):
        pltpu.make_async_copy(out.at[pl.ds(row0, CH)], buf, sem).wait()

    g_tok(0, tok_buf0, sem_t0)
    g_tok(1, tok_buf1, sem_t1)
    w_tok(0, tok_buf0, sem_t0)
    compute_chunk(0, tok_buf0)
    put_out(0, tok_buf0, sem_o0)

    def pipe_step(cc, carry):
        c1 = 2 * cc + 1
        c2 = 2 * cc + 2
        w_tok(c1, tok_buf1, sem_t1)
        w_out(tok_buf0, sem_o0)
        g_tok(c2, tok_buf0, sem_t0)
        compute_chunk(c1, tok_buf1)
        put_out(c1, tok_buf1, sem_o1)
        c3 = c2 + 1
        w_tok(c2, tok_buf0, sem_t0)
        w_out(tok_buf1, sem_o1)
        g_tok(c3, tok_buf1, sem_t1)
        compute_chunk(c2, tok_buf0)
        put_out(c2, tok_buf0, sem_o0)
        return carry
    lax.fori_loop(0, (NCH - 2) // 2, pipe_step, 0)

    cl = NCH - 1
    w_tok(cl, tok_buf1, sem_t1)
    compute_chunk(cl, tok_buf1)
    put_out(cl, tok_buf1, sem_o1)
    w_out(tok_buf0, sem_o0)
    w_out(tok_buf1, sem_o1)


def kernel(input_ids, note_type_ids, concentration_ids, position_ids,
           season_ids, emotion_ids, time_ids,
           token_table, note_table, conc_table, pos_table,
           season_table, emotion_table, time_table):
    tok_ids = input_ids.reshape(N // CH, CH).astype(jnp.int32)
    pos_ids2 = position_ids.reshape(N // CH, CH).astype(jnp.int32)
    note_ids2 = note_type_ids.reshape(N // CH, CH).astype(jnp.int32)
    conc_ids2 = concentration_ids.reshape(N // CH, CH).astype(jnp.int32)
    sids = season_ids.astype(jnp.int32)
    eids = emotion_ids.astype(jnp.int32)
    tids = time_ids.astype(jnp.int32)

    mesh = plsc.VectorSubcoreMesh(
        core_axis_name="c", subcore_axis_name="s",
        num_cores=NC, num_subcores=NS)
    run = pl.kernel(
        _body,
        out_type=jax.ShapeDtypeStruct((N, D), jnp.float32),
        mesh=mesh,
        compiler_params=pltpu.CompilerParams(
            use_tc_tiling_on_sc=False, needs_layout_passes=False),
        scratch_types=[
            pltpu.VMEM((NCH, CH), jnp.int32),
            pltpu.VMEM((NCH, CH), jnp.int32),
            pltpu.VMEM((NCH, CH), jnp.int32),
            pltpu.VMEM((NCH, CH), jnp.int32),
            pltpu.VMEM((3, D), BF),
            pltpu.VMEM((20, D), BF),
            pltpu.VMEM((4, D), BF),
            pltpu.VMEM((8, D), BF),
            pltpu.VMEM((4, D), BF),
            pltpu.VMEM((60, D // 2), jnp.int32),
            pltpu.VMEM((BPW, D), BF),
            pltpu.VMEM((BPW,), jnp.int32),
            pltpu.VMEM((BPW,), jnp.int32),
            pltpu.VMEM((BPW,), jnp.int32),
            pltpu.VMEM((512, D // 2), jnp.int32),
            pltpu.VMEM((CH, D), jnp.float32),
            pltpu.VMEM((CH, D), jnp.float32),
            pltpu.SemaphoreType.DMA,
            pltpu.SemaphoreType.DMA,
            pltpu.SemaphoreType.DMA,
            pltpu.SemaphoreType.DMA,
        ],
    )
    perm = (jnp.arange(0, D, 2 * L)[:, None]
            + jnp.stack([jnp.arange(L), jnp.arange(L) + L], 1).reshape(-1)
            ).reshape(-1)

    def pack_i32(t):
        tb = t[:, perm].astype(BF)
        return lax.bitcast_convert_type(
            tb.reshape(t.shape[0], t.shape[1] // 2, 2), jnp.int32)

    def perm_bf(t):
        return t[:, perm].astype(BF)

    out = run(tok_ids, pos_ids2, note_ids2, conc_ids2, sids, eids, tids,
              token_table, perm_bf(note_table),
              perm_bf(conc_table), pack_i32(pos_table),
              perm_bf(season_table), perm_bf(emotion_table),
              perm_bf(time_table))
    return out.reshape(B, S, D)
